# Initial kernel scaffold; baseline (speedup 1.0000x reference)
import jax
import jax.numpy as jnp
from jax.experimental import pallas as pl


def kernel(X, idx, attr, batch, params):
    raise NotImplementedError("write your pallas kernel here")


# speedup vs baseline: 10.4065x; 10.4065x over previous
import jax
import jax.numpy as jnp
from jax import lax
from jax.experimental import pallas as pl
from jax.experimental.pallas import tpu as pltpu
from jax.experimental.pallas import tpu_sc as plsc

H = 5
C = 32
HC = H * C
NG = 16
N = 50000
E = 800000
NPAD = 50176
R = 16
RS = NPAD // R
ACC_ROWS = RS + 64
NSUB = 16
EPT = E // NSUB
CH = 2000
NCH = EPT // CH
B = 64
PCAP = 6464
MSG_W = HC + 16
INV_SQRT_C = 1.0 / (C ** 0.5)


def _mp_body(qh, kh, vh, eh, srcv, dstv, out,
             pend_l, pend_s, pend_e, dbuf, sbuf,
             blk_g, blk_l, blk_s, blk_e,
             qbuf, kbuf, vbuf, ebuf, wvbuf, wbuf, acc,
             sem_q, sem_k, sem_v, sem_e):
    c = lax.axis_index("c")
    s = lax.axis_index("s")
    ebase = s * EPT
    iota = jnp.arange(16, dtype=jnp.int32)
    zeros16 = jnp.zeros((16,), jnp.float32)
    ones16i = jnp.ones((16,), jnp.int32)

    def sweep_body(sweep, _):
        nb = (sweep * 2 + c) * RS

        def zw(i, _):
            wvbuf[i // 11, pl.ds((i % 11) * 16, 16)] = zeros16
            return 0
        lax.fori_loop(0, B * 11, zw, 0)
        zbase = s * (ACC_ROWS // 16)
        for j in range(3):
            pltpu.sync_copy(wvbuf.at[pl.ds(0, B)],
                            acc.at[pl.ds(zbase + j * B, B)])
        pltpu.sync_copy(wvbuf.at[pl.ds(0, 8)],
                        acc.at[pl.ds(zbase + 192, 8)])
        plsc.subcore_barrier()

        def chunk_body(ci, cntv):
            off = ebase + ci * CH
            pltpu.sync_copy(dstv.at[pl.ds(off, CH)], dbuf)
            pltpu.sync_copy(srcv.at[pl.ds(off, CH)], sbuf)

            def g_body(g, cntv):
                dvec = dbuf[pl.ds(g * 16, 16)]
                svec = sbuf[pl.ds(g * 16, 16)]
                evec = off + g * 16 + iota
                m = (dvec >= nb) & (dvec < nb + RS)
                pref = plsc.cumsum(ones16i, mask=m)
                pos = cntv + pref - 1
                plsc.store_scatter(pend_l, [pos], dvec - nb, mask=m)
                plsc.store_scatter(pend_s, [pos], svec, mask=m)
                plsc.store_scatter(pend_e, [pos], evec, mask=m)
                return cntv + plsc.all_reduce_population_count(m)

            return lax.fori_loop(0, CH // 16, g_body, cntv)

        cntv = lax.fori_loop(0, NCH, chunk_body, jnp.zeros((16,), jnp.int32))
        cnt = cntv[0]

        padl = jnp.full((16,), RS, jnp.int32)
        padz = jnp.zeros((16,), jnp.int32)
        for t in range(B // 16):
            pend_l[pl.ds(cnt + t * 16, 16)] = padl
            pend_s[pl.ds(cnt + t * 16, 16)] = padz
            pend_e[pl.ds(cnt + t * 16, 16)] = padz
        nflush = (cnt + B - 1) // B

        def flush_body(b, _):
            off = b * B
            for j in range(B // 16):
                plv = pend_l[pl.ds(off + j * 16, 16)]
                blk_g[pl.ds(j * 16, 16)] = jnp.where(plv >= RS, 0, plv) + nb
                blk_l[pl.ds(j * 16, 16)] = plv
                blk_s[pl.ds(j * 16, 16)] = pend_s[pl.ds(off + j * 16, 16)]
                blk_e[pl.ds(j * 16, 16)] = pend_e[pl.ds(off + j * 16, 16)]
            cq = pltpu.async_copy(qh.at[blk_g], qbuf, sem_q)
            ck = pltpu.async_copy(kh.at[blk_s], kbuf, sem_k)
            cv = pltpu.async_copy(vh.at[blk_s], vbuf, sem_v)
            ce = pltpu.async_copy(eh.at[blk_e], ebuf, sem_e)
            cq.wait()
            ck.wait()
            cv.wait()
            ce.wait()

            def grp(j, _):
                rows = j * 16 + iota
                for h in range(H):
                    a = zeros16
                    for cc in range(C):
                        cols = jnp.full((16,), h * C + cc, jnp.int32)
                        kv = plsc.load_gather(kbuf, [rows, cols])
                        ev = plsc.load_gather(ebuf, [rows, cols])
                        qv = plsc.load_gather(qbuf, [rows, cols])
                        a = a + qv * (kv + ev)
                    w = jnp.exp(a * INV_SQRT_C)
                    wbuf[pl.ds(h * B + j * 16, 16)] = w
                return 0
            lax.fori_loop(0, B // 16, grp, 0)

            def wrow(b2, _):
                den = zeros16
                for h in range(H):
                    wb = plsc.load_gather(
                        wbuf, [jnp.full((16,), h * B, jnp.int32) + b2])
                    den = den + jnp.where(iota == h, wb, 0.0)
                    for cc in range(2):
                        co = h * C + cc * 16
                        vv = vbuf[b2, pl.ds(co, 16)] + ebuf[b2, pl.ds(co, 16)]
                        wvbuf[b2, pl.ds(co, 16)] = wb * vv
                wvbuf[b2, pl.ds(HC, 16)] = den
                return 0
            lax.fori_loop(0, B, wrow, 0)

            pltpu.sync_copy(wvbuf, acc.at[blk_l], add=True)
            return 0

        lax.fori_loop(0, nflush, flush_body, 0)
        plsc.subcore_barrier()

        @pl.when(s < 8)
        def _():
            pltpu.sync_copy(acc.at[pl.ds(s * 392, 392)],
                            out.at[pl.ds(nb + s * 392, 392)])
        plsc.subcore_barrier()
        return 0

    lax.fori_loop(0, R // 2, sweep_body, 0)


def _make_mp():
    mesh = plsc.VectorSubcoreMesh(core_axis_name="c", subcore_axis_name="s")
    return pl.kernel(
        _mp_body,
        out_type=jax.ShapeDtypeStruct((NPAD, MSG_W), jnp.float32),
        mesh=mesh,
        compiler_params=pltpu.CompilerParams(needs_layout_passes=False, use_tc_tiling_on_sc=False),
        scratch_types=[
            pltpu.VMEM((PCAP,), jnp.int32),
            pltpu.VMEM((PCAP,), jnp.int32),
            pltpu.VMEM((PCAP,), jnp.int32),
            pltpu.VMEM((CH,), jnp.int32),
            pltpu.VMEM((CH,), jnp.int32),
            pltpu.VMEM((B,), jnp.int32),
            pltpu.VMEM((B,), jnp.int32),
            pltpu.VMEM((B,), jnp.int32),
            pltpu.VMEM((B,), jnp.int32),
            pltpu.VMEM((B, HC), jnp.float32),
            pltpu.VMEM((B, HC), jnp.float32),
            pltpu.VMEM((B, HC), jnp.float32),
            pltpu.VMEM((B, HC), jnp.float32),
            pltpu.VMEM((B, MSG_W), jnp.float32),
            pltpu.VMEM((H * B,), jnp.float32),
            pltpu.VMEM_SHARED((ACC_ROWS, MSG_W), jnp.float32),
            pltpu.SemaphoreType.DMA,
            pltpu.SemaphoreType.DMA,
            pltpu.SemaphoreType.DMA,
            pltpu.SemaphoreType.DMA,
        ],
    )


BN = 1792
NBLK = NPAD // BN
BE = 4000


def _proj_fn(x_ref, w_ref, b_ref, q_ref, k_ref, v_ref, s_ref):
    y = jnp.dot(x_ref[...], w_ref[...],
                preferred_element_type=jnp.float32) + b_ref[...]
    q_ref[...] = y[:, 0 * HC:1 * HC]
    k_ref[...] = y[:, 1 * HC:2 * HC]
    v_ref[...] = y[:, 2 * HC:3 * HC]
    s_ref[...] = y[:, 3 * HC:4 * HC]


def _proj(x, w4, b4):
    k = x.shape[1]
    outs = [jax.ShapeDtypeStruct((NPAD, HC), jnp.float32)] * 4
    return pl.pallas_call(
        _proj_fn,
        grid=(NBLK,),
        in_specs=[
            pl.BlockSpec((BN, k), lambda i: (i, 0)),
            pl.BlockSpec((k, 4 * HC), lambda i: (0, 0)),
            pl.BlockSpec((1, 4 * HC), lambda i: (0, 0)),
        ],
        out_specs=[pl.BlockSpec((BN, HC), lambda i: (i, 0))] * 4,
        out_shape=outs,
    )(x, w4, b4)


def _eproj_fn(a_ref, w_ref, o_ref):
    o_ref[...] = jnp.dot(a_ref[...], w_ref[...],
                         preferred_element_type=jnp.float32)


def _eproj(attr, we):
    return pl.pallas_call(
        _eproj_fn,
        grid=(E // BE,),
        in_specs=[
            pl.BlockSpec((BE, 16), lambda i: (i, 0)),
            pl.BlockSpec((16, HC), lambda i: (0, 0)),
        ],
        out_specs=pl.BlockSpec((BE, HC), lambda i: (i, 0)),
        out_shape=jax.ShapeDtypeStruct((E, HC), jnp.float32),
    )(attr, we)


def _combine_fn(msg_ref, skip_ref, x_ref, st_ref):
    pid = pl.program_id(0)
    m = msg_ref[...]
    den = m[:, HC:HC + H]
    den = jnp.repeat(den, C, axis=1)
    x = m[:, :HC] / (den + 1e-16) + skip_ref[...]
    row = pid * BN + lax.broadcasted_iota(jnp.int32, (BN, 1), 0)
    x = jnp.where(row < N, x, 0.0)
    x_ref[...] = x

    @pl.when(pid == 0)
    def _():
        st_ref[...] = jnp.zeros_like(st_ref)
    upd = jnp.concatenate(
        [jnp.sum(x, axis=0, keepdims=True),
         jnp.sum(x * x, axis=0, keepdims=True),
         jnp.zeros((6, HC), jnp.float32)], axis=0)
    st_ref[...] += upd


def _combine(msg, skip):
    return pl.pallas_call(
        _combine_fn,
        grid=(NBLK,),
        in_specs=[
            pl.BlockSpec((BN, MSG_W), lambda i: (i, 0)),
            pl.BlockSpec((BN, HC), lambda i: (i, 0)),
        ],
        out_specs=[
            pl.BlockSpec((BN, HC), lambda i: (i, 0)),
            pl.BlockSpec((8, HC), lambda i: (0, 0)),
        ],
        out_shape=[
            jax.ShapeDtypeStruct((NPAD, HC), jnp.float32),
            jax.ShapeDtypeStruct((8, HC), jnp.float32),
        ],
    )(msg, skip)


def _apply_fn(x_ref, st_ref, g_ref, b_ref, o_ref):
    pid = pl.program_id(0)
    st = st_ref[...]
    mean = st[0:1, :] / float(N)
    var = st[1:2, :] / float(N) - mean * mean
    rstd = lax.rsqrt(var + 1e-5)
    y = g_ref[...] * (x_ref[...] - mean) * rstd + b_ref[...]
    y = jnp.maximum(y, 0.0)
    row = pid * BN + lax.broadcasted_iota(jnp.int32, (BN, 1), 0)
    o_ref[...] = jnp.where(row < N, y, 0.0)


def _apply(x, st, g, b):
    return pl.pallas_call(
        _apply_fn,
        grid=(NBLK,),
        in_specs=[
            pl.BlockSpec((BN, HC), lambda i: (i, 0)),
            pl.BlockSpec((8, HC), lambda i: (0, 0)),
            pl.BlockSpec((1, HC), lambda i: (0, 0)),
            pl.BlockSpec((1, HC), lambda i: (0, 0)),
        ],
        out_specs=pl.BlockSpec((BN, HC), lambda i: (i, 0)),
        out_shape=jax.ShapeDtypeStruct((NPAD, HC), jnp.float32),
    )(x, st, g, b)


def _pool_fn(x1_ref, x2_ref, x3_ref, b_ref, s_ref):
    pid = pl.program_id(0)
    bv = b_ref[0, 0, :]
    oh = (lax.broadcasted_iota(jnp.int32, (NG, BN), 0)
          == bv[None, :]).astype(jnp.float32)
    xc = jnp.concatenate(
        [x1_ref[...], x2_ref[...], x3_ref[...],
         jnp.ones((BN, 128), jnp.float32)], axis=1)
    acc = jnp.dot(oh, xc, preferred_element_type=jnp.float32)

    @pl.when(pid == 0)
    def _():
        s_ref[...] = jnp.zeros_like(s_ref)
    s_ref[...] += acc


def _pool(x1, x2, x3, batch3d):
    return pl.pallas_call(
        _pool_fn,
        grid=(NBLK,),
        in_specs=[
            pl.BlockSpec((BN, HC), lambda i: (i, 0)),
            pl.BlockSpec((BN, HC), lambda i: (i, 0)),
            pl.BlockSpec((BN, HC), lambda i: (i, 0)),
            pl.BlockSpec((1, 1, BN), lambda i: (i, 0, 0)),
        ],
        out_specs=pl.BlockSpec((NG, 3 * HC + 128), lambda i: (0, 0)),
        out_shape=jax.ShapeDtypeStruct((NG, 3 * HC + 128), jnp.float32),
    )(x1, x2, x3, batch3d)


def _final_fn(s_ref, w_ref, b_ref, g_ref, bb_ref, o_ref):
    srow = s_ref[...]
    cnt = jnp.maximum(srow[:, 3 * HC:3 * HC + 1], 1.0)
    pooled = srow[:, :3 * HC] / cnt
    m = jnp.mean(pooled, axis=0, keepdims=True)
    v = jnp.mean((pooled - m) ** 2, axis=0, keepdims=True)
    pn = g_ref[...] * (pooled - m) * lax.rsqrt(v + 1e-5) + bb_ref[...]
    o_ref[...] = jnp.dot(pn, w_ref[...],
                         preferred_element_type=jnp.float32) + b_ref[...]


def _final(sums, fcw, fcb, g, b):
    return pl.pallas_call(
        _final_fn,
        in_specs=[pl.BlockSpec(sums.shape, lambda: (0, 0)),
                  pl.BlockSpec(fcw.shape, lambda: (0, 0)),
                  pl.BlockSpec(fcb.shape, lambda: (0, 0)),
                  pl.BlockSpec(g.shape, lambda: (0, 0)),
                  pl.BlockSpec(b.shape, lambda: (0, 0))],
        out_specs=pl.BlockSpec((NG, 64), lambda: (0, 0)),
        out_shape=jax.ShapeDtypeStruct((NG, 64), jnp.float32),
    )(sums, fcw, fcb, g, b)


def kernel(X, idx, attr, batch, params):
    mp = _make_mp()

    src = idx[0]
    dst = idx[1]
    x = jnp.pad(X, ((0, NPAD - N), (0, 64 - X.shape[1])))
    batchp = jnp.pad(batch, (0, NPAD - N), constant_values=NG)
    batch3d = batchp.reshape(NBLK, 1, BN)

    xs = []
    for ln in ("l1", "l2", "l3"):
        p = params[ln]
        fin = p["Wq"].shape[0]
        kpad = 64 if fin == 55 else fin
        w4 = jnp.concatenate([p["Wq"], p["Wk"], p["Wv"], p["Wskip"]], axis=1)
        w4 = jnp.pad(w4, ((0, kpad - fin), (0, 0)))
        b4 = jnp.concatenate([p["bq"], p["bk"], p["bv"], p["bskip"]])
        b4 = b4.reshape(1, 4 * HC)
        q, k, v, skip = _proj(x, w4, b4)
        e = _eproj(attr, p["We"])
        msg = mp(q, k, v, e, src, dst)
        li = len(xs) + 1
        xcomb, st = _combine(msg, skip)
        x = _apply(xcomb, st,
                   params[f"bn{li}_g"].reshape(1, HC),
                   params[f"bn{li}_b"].reshape(1, HC))
        xs.append(x)

    sums = _pool(xs[0], xs[1], xs[2], batch3d)
    out = _final(sums, params["fc_W"],
                 params["fc_b"].reshape(1, 64),
                 params["bn_out_g"].reshape(1, 3 * HC),
                 params["bn_out_b"].reshape(1, 3 * HC))
    return out


# speedup vs baseline: 21.2988x; 2.0467x over previous
import jax
import jax.numpy as jnp
from jax import lax
from jax.experimental import pallas as pl
from jax.experimental.pallas import tpu as pltpu
from jax.experimental.pallas import tpu_sc as plsc

H = 5
C = 32
HC = H * C
NG = 16
N = 50000
E = 800000
NPAD = 50176
R = 16
RS = NPAD // R
ACC_ROWS = RS + 64
NSUB = 16
EPT = 51200
EPAD = NSUB * EPT
CH = 1600
NCH = EPT // CH
B = 32
PCAP = 6464
MSG_W = HC + 16
INV_SQRT_C = 1.0 / (C ** 0.5)


def _mp_body(qh, kvh, eh, srcv, dstv, out,
             pend_l, pend_s, pend_e,
             dbuf0, dbuf1, sbuf0, sbuf1,
             blk_g0, blk_g1, blk_l0, blk_l1, blk_s0, blk_s1,
             blk_e0, blk_e1, sblk0, sblk1,
             qbuf0, qbuf1, kvbuf0, kvbuf1, ebuf0, ebuf1,
             wvbuf0, wvbuf1, wbuf, acc,
             sem_g0, sem_g1, sem_s0, sem_s1, sem_c0, sem_c1):
    c = lax.axis_index("c")
    s = lax.axis_index("s")
    ebase = s * EPT
    iota = jnp.arange(16, dtype=jnp.int32)
    zeros16 = jnp.zeros((16,), jnp.float32)
    ones16i = jnp.ones((16,), jnp.int32)
    dbuf = (dbuf0, dbuf1)
    sbuf = (sbuf0, sbuf1)
    blk_g = (blk_g0, blk_g1)
    blk_l = (blk_l0, blk_l1)
    blk_s = (blk_s0, blk_s1)
    blk_e = (blk_e0, blk_e1)
    sblk = (sblk0, sblk1)
    qbuf = (qbuf0, qbuf1)
    kvbuf = (kvbuf0, kvbuf1)
    ebuf = (ebuf0, ebuf1)
    wvbuf = (wvbuf0, wvbuf1)
    sem_g = (sem_g0, sem_g1)
    sem_s = (sem_s0, sem_s1)
    sem_c = (sem_c0, sem_c1)

    def fire_chunk(ci, p):
        off = ebase + ci * CH
        pltpu.async_copy(dstv.at[pl.ds(off, CH)], dbuf[p], sem_c[p])
        pltpu.async_copy(srcv.at[pl.ds(off, CH)], sbuf[p], sem_c[p])

    def wait_chunk(ci, p):
        off = ebase + ci * CH
        pltpu.make_async_copy(dstv.at[pl.ds(off, CH)], dbuf[p], sem_c[p]).wait()
        pltpu.make_async_copy(srcv.at[pl.ds(off, CH)], sbuf[p], sem_c[p]).wait()

    def sweep_body(sweep, _):
        nb = (sweep * 2 + c) * RS

        def zw(i, _):
            wvbuf0[i // 11, pl.ds((i % 11) * 16, 16)] = zeros16
            return 0
        lax.fori_loop(0, B * 11, zw, 0)
        zbase = s * (ACC_ROWS // 16)
        for j in range(6):
            pltpu.sync_copy(wvbuf0.at[pl.ds(0, B)],
                            acc.at[pl.ds(zbase + j * B, B)])
        pltpu.sync_copy(wvbuf0.at[pl.ds(0, 8)],
                        acc.at[pl.ds(zbase + 192, 8)])
        plsc.subcore_barrier()

        def g_factory(p, off):
            def g_body(g, cntv):
                dvec = dbuf[p][pl.ds(g * 16, 16)]
                svec = sbuf[p][pl.ds(g * 16, 16)]
                evec = off + g * 16 + iota
                m = (dvec >= nb) & (dvec < nb + RS)
                pref = plsc.cumsum(ones16i, mask=m)
                pos = cntv + pref - 1
                plsc.store_scatter(pend_l, [pos], dvec - nb, mask=m)
                plsc.store_scatter(pend_s, [pos], svec, mask=m)
                plsc.store_scatter(pend_e, [pos], evec, mask=m)
                return cntv + plsc.all_reduce_population_count(m)
            return g_body

        fire_chunk(0, 0)

        def chunk_pair(it, cntv):
            for p in range(2):
                ci = 2 * it + p
                wait_chunk(ci, p)

                @pl.when(ci + 1 < NCH)
                def _():
                    fire_chunk(ci + 1, 1 - p)
                cntv = lax.fori_loop(0, CH // 16,
                                     g_factory(p, ebase + ci * CH), cntv)
            return cntv

        cntv = lax.fori_loop(0, NCH // 2, chunk_pair,
                             jnp.zeros((16,), jnp.int32))
        cnt = cntv[0]

        padl = jnp.full((16,), RS, jnp.int32)
        padz = jnp.zeros((16,), jnp.int32)
        for t in range(4):
            pend_l[pl.ds(cnt + t * 16, 16)] = padl
            pend_s[pl.ds(cnt + t * 16, 16)] = padz
            pend_e[pl.ds(cnt + t * 16, 16)] = padz
        nblocks = jnp.maximum(2, ((cnt + 2 * B - 1) // (2 * B)) * 2)

        def prep_fire(b, p):
            off = b * B
            for j in range(B // 16):
                plv = pend_l[pl.ds(off + j * 16, 16)]
                psv = pend_s[pl.ds(off + j * 16, 16)]
                pev = pend_e[pl.ds(off + j * 16, 16)]
                plc = jnp.clip(plv, 0, RS)
                blk_g[p][pl.ds(j * 16, 16)] = jnp.where(plc >= RS, 0, plc) + nb
                blk_l[p][pl.ds(j * 16, 16)] = plc
                blk_s[p][pl.ds(j * 16, 16)] = jnp.clip(psv, 0, NPAD - 1)
                blk_e[p][pl.ds(j * 16, 16)] = jnp.clip(pev, 0, E - 1)
            pltpu.async_copy(qh.at[blk_g[p]], qbuf[p], sem_g[p])
            pltpu.async_copy(kvh.at[blk_s[p]], kvbuf[p], sem_g[p])
            pltpu.async_copy(eh.at[blk_e[p]], ebuf[p], sem_g[p])

        m15 = iota == 15

        def compute(p):
            def arow(b2, _):
                for h in range(H):
                    co = h * C
                    u = (qbuf[p][b2, pl.ds(co, 16)]
                         * (kvbuf[p][b2, pl.ds(co, 16)]
                            + ebuf[p][b2, pl.ds(co, 16)])
                         + qbuf[p][b2, pl.ds(co + 16, 16)]
                         * (kvbuf[p][b2, pl.ds(co + 16, 16)]
                            + ebuf[p][b2, pl.ds(co + 16, 16)]))
                    cs = plsc.cumsum(u)
                    plsc.store_scatter(
                        wbuf, [jnp.full((16,), h * B, jnp.int32) + b2], cs,
                        mask=m15)
                return 0
            lax.fori_loop(0, B, arow, 0)
            for i in range(H * B // 16):
                av = wbuf[pl.ds(i * 16, 16)]
                wbuf[pl.ds(i * 16, 16)] = jnp.exp(av * INV_SQRT_C)

            def wrow(b2, _):
                den = zeros16
                for h in range(H):
                    wb = plsc.load_gather(
                        wbuf, [jnp.full((16,), h * B, jnp.int32) + b2])
                    den = den + jnp.where(iota == h, wb, 0.0)
                    for cc in range(2):
                        co = h * C + cc * 16
                        vv = (kvbuf[p][b2, pl.ds(HC + co, 16)]
                              + ebuf[p][b2, pl.ds(co, 16)])
                        wvbuf[p][b2, pl.ds(co, 16)] = wb * vv
                wvbuf[p][b2, pl.ds(HC, 16)] = den
                return 0
            lax.fori_loop(0, B, wrow, 0)

        prep_fire(0, 0)
        prep_fire(1, 1)

        def flush_pair(it, _):
            for p in range(2):
                b = 2 * it + p
                pltpu.make_async_copy(qh.at[blk_g[p]], qbuf[p],
                                      sem_g[p]).wait()
                pltpu.make_async_copy(kvh.at[blk_s[p]], kvbuf[p],
                                      sem_g[p]).wait()
                pltpu.make_async_copy(eh.at[blk_e[p]], ebuf[p],
                                      sem_g[p]).wait()

                @pl.when(it >= 1)
                def _():
                    pltpu.make_async_copy(wvbuf[p], acc.at[sblk[p]],
                                          sem_s[p]).wait()
                compute(p)
                for j in range(B // 16):
                    sblk[p][pl.ds(j * 16, 16)] = blk_l[p][pl.ds(j * 16, 16)]
                pltpu.async_copy(wvbuf[p], acc.at[sblk[p]], sem_s[p],
                                 add=True)

                @pl.when(b + 2 < nblocks)
                def _():
                    prep_fire(b + 2, p)
            return 0

        lax.fori_loop(0, nblocks // 2, flush_pair, 0)
        for p in range(2):
            pltpu.make_async_copy(wvbuf[p], acc.at[sblk[p]], sem_s[p]).wait()
        plsc.subcore_barrier()

        @pl.when(s < 8)
        def _():
            pltpu.sync_copy(acc.at[pl.ds(s * 392, 392)],
                            out.at[pl.ds(nb + s * 392, 392)])
        plsc.subcore_barrier()
        return 0

    lax.fori_loop(0, R // 2, sweep_body, 0)


def _make_mp():
    mesh = plsc.VectorSubcoreMesh(core_axis_name="c", subcore_axis_name="s")
    i32 = jnp.int32
    f32 = jnp.float32
    return pl.kernel(
        _mp_body,
        out_type=jax.ShapeDtypeStruct((NPAD, MSG_W), f32),
        mesh=mesh,
        compiler_params=pltpu.CompilerParams(needs_layout_passes=False,
                                             use_tc_tiling_on_sc=False),
        scratch_types=(
            [pltpu.VMEM((PCAP,), i32)] * 3
            + [pltpu.VMEM((CH,), i32)] * 4
            + [pltpu.VMEM((B,), i32)] * 10
            + [pltpu.VMEM((B, HC), f32)] * 2
            + [pltpu.VMEM((B, 2 * HC), f32)] * 2
            + [pltpu.VMEM((B, HC), f32)] * 2
            + [pltpu.VMEM((B, MSG_W), f32)] * 2
            + [pltpu.VMEM((H * B,), f32)]
            + [pltpu.VMEM_SHARED((ACC_ROWS, MSG_W), f32)]
            + [pltpu.SemaphoreType.DMA] * 6
        ),
    )


BN = 1792
NBLK = NPAD // BN
BE = 4000


def _proj_fn(x_ref, w_ref, b_ref, q_ref, kv_ref, s_ref):
    y = jnp.dot(x_ref[...], w_ref[...],
                preferred_element_type=jnp.float32) + b_ref[...]
    q_ref[...] = y[:, 0 * HC:1 * HC]
    kv_ref[...] = y[:, 1 * HC:3 * HC]
    s_ref[...] = y[:, 3 * HC:4 * HC]


def _proj(x, w4, b4):
    k = x.shape[1]
    outs = [jax.ShapeDtypeStruct((NPAD, HC), jnp.float32),
            jax.ShapeDtypeStruct((NPAD, 2 * HC), jnp.float32),
            jax.ShapeDtypeStruct((NPAD, HC), jnp.float32)]
    return pl.pallas_call(
        _proj_fn,
        grid=(NBLK,),
        in_specs=[
            pl.BlockSpec((BN, k), lambda i: (i, 0)),
            pl.BlockSpec((k, 4 * HC), lambda i: (0, 0)),
            pl.BlockSpec((1, 4 * HC), lambda i: (0, 0)),
        ],
        out_specs=[pl.BlockSpec((BN, HC), lambda i: (i, 0)),
                   pl.BlockSpec((BN, 2 * HC), lambda i: (i, 0)),
                   pl.BlockSpec((BN, HC), lambda i: (i, 0))],
        out_shape=outs,
    )(x, w4, b4)


def _eproj_fn(a_ref, w_ref, o_ref):
    o_ref[...] = jnp.dot(a_ref[...], w_ref[...],
                         preferred_element_type=jnp.float32)


def _eproj(attr, we):
    return pl.pallas_call(
        _eproj_fn,
        grid=(E // BE,),
        in_specs=[
            pl.BlockSpec((BE, 16), lambda i: (i, 0)),
            pl.BlockSpec((16, HC), lambda i: (0, 0)),
        ],
        out_specs=pl.BlockSpec((BE, HC), lambda i: (i, 0)),
        out_shape=jax.ShapeDtypeStruct((E, HC), jnp.float32),
    )(attr, we)


def _combine_fn(msg_ref, skip_ref, x_ref, st_ref):
    pid = pl.program_id(0)
    m = msg_ref[...]
    den = m[:, HC:HC + H]
    den = jnp.repeat(den, C, axis=1)
    x = m[:, :HC] / (den + 1e-16) + skip_ref[...]
    row = pid * BN + lax.broadcasted_iota(jnp.int32, (BN, 1), 0)
    x = jnp.where(row < N, x, 0.0)
    x_ref[...] = x

    @pl.when(pid == 0)
    def _():
        st_ref[...] = jnp.zeros_like(st_ref)
    upd = jnp.concatenate(
        [jnp.sum(x, axis=0, keepdims=True),
         jnp.sum(x * x, axis=0, keepdims=True),
         jnp.zeros((6, HC), jnp.float32)], axis=0)
    st_ref[...] += upd


def _combine(msg, skip):
    return pl.pallas_call(
        _combine_fn,
        grid=(NBLK,),
        in_specs=[
            pl.BlockSpec((BN, MSG_W), lambda i: (i, 0)),
            pl.BlockSpec((BN, HC), lambda i: (i, 0)),
        ],
        out_specs=[
            pl.BlockSpec((BN, HC), lambda i: (i, 0)),
            pl.BlockSpec((8, HC), lambda i: (0, 0)),
        ],
        out_shape=[
            jax.ShapeDtypeStruct((NPAD, HC), jnp.float32),
            jax.ShapeDtypeStruct((8, HC), jnp.float32),
        ],
    )(msg, skip)


def _apply_fn(x_ref, st_ref, g_ref, b_ref, o_ref):
    pid = pl.program_id(0)
    st = st_ref[...]
    mean = st[0:1, :] / float(N)
    var = st[1:2, :] / float(N) - mean * mean
    rstd = lax.rsqrt(var + 1e-5)
    y = g_ref[...] * (x_ref[...] - mean) * rstd + b_ref[...]
    y = jnp.maximum(y, 0.0)
    row = pid * BN + lax.broadcasted_iota(jnp.int32, (BN, 1), 0)
    o_ref[...] = jnp.where(row < N, y, 0.0)


def _apply(x, st, g, b):
    return pl.pallas_call(
        _apply_fn,
        grid=(NBLK,),
        in_specs=[
            pl.BlockSpec((BN, HC), lambda i: (i, 0)),
            pl.BlockSpec((8, HC), lambda i: (0, 0)),
            pl.BlockSpec((1, HC), lambda i: (0, 0)),
            pl.BlockSpec((1, HC), lambda i: (0, 0)),
        ],
        out_specs=pl.BlockSpec((BN, HC), lambda i: (i, 0)),
        out_shape=jax.ShapeDtypeStruct((NPAD, HC), jnp.float32),
    )(x, st, g, b)


def _pool_fn(x1_ref, x2_ref, x3_ref, b_ref, s_ref):
    pid = pl.program_id(0)
    bv = b_ref[0, 0, :]
    oh = (lax.broadcasted_iota(jnp.int32, (NG, BN), 0)
          == bv[None, :]).astype(jnp.float32)
    xc = jnp.concatenate(
        [x1_ref[...], x2_ref[...], x3_ref[...],
         jnp.ones((BN, 128), jnp.float32)], axis=1)
    acc = jnp.dot(oh, xc, preferred_element_type=jnp.float32)

    @pl.when(pid == 0)
    def _():
        s_ref[...] = jnp.zeros_like(s_ref)
    s_ref[...] += acc


def _pool(x1, x2, x3, batch3d):
    return pl.pallas_call(
        _pool_fn,
        grid=(NBLK,),
        in_specs=[
            pl.BlockSpec((BN, HC), lambda i: (i, 0)),
            pl.BlockSpec((BN, HC), lambda i: (i, 0)),
            pl.BlockSpec((BN, HC), lambda i: (i, 0)),
            pl.BlockSpec((1, 1, BN), lambda i: (i, 0, 0)),
        ],
        out_specs=pl.BlockSpec((NG, 3 * HC + 128), lambda i: (0, 0)),
        out_shape=jax.ShapeDtypeStruct((NG, 3 * HC + 128), jnp.float32),
    )(x1, x2, x3, batch3d)


def _final_fn(s_ref, w_ref, b_ref, g_ref, bb_ref, o_ref):
    srow = s_ref[...]
    cnt = jnp.maximum(srow[:, 3 * HC:3 * HC + 1], 1.0)
    pooled = srow[:, :3 * HC] / cnt
    m = jnp.mean(pooled, axis=0, keepdims=True)
    v = jnp.mean((pooled - m) ** 2, axis=0, keepdims=True)
    pn = g_ref[...] * (pooled - m) * lax.rsqrt(v + 1e-5) + bb_ref[...]
    o_ref[...] = jnp.dot(pn, w_ref[...],
                         preferred_element_type=jnp.float32) + b_ref[...]


def _final(sums, fcw, fcb, g, b):
    return pl.pallas_call(
        _final_fn,
        in_specs=[pl.BlockSpec(sums.shape, lambda: (0, 0)),
                  pl.BlockSpec(fcw.shape, lambda: (0, 0)),
                  pl.BlockSpec(fcb.shape, lambda: (0, 0)),
                  pl.BlockSpec(g.shape, lambda: (0, 0)),
                  pl.BlockSpec(b.shape, lambda: (0, 0))],
        out_specs=pl.BlockSpec((NG, 64), lambda: (0, 0)),
        out_shape=jax.ShapeDtypeStruct((NG, 64), jnp.float32),
    )(sums, fcw, fcb, g, b)


def kernel(X, idx, attr, batch, params):
    mp = _make_mp()

    src = jnp.pad(idx[0], (0, EPAD - E))
    dst = jnp.pad(idx[1], (0, EPAD - E), constant_values=-1)
    x = jnp.pad(X, ((0, NPAD - N), (0, 64 - X.shape[1])))
    batchp = jnp.pad(batch, (0, NPAD - N), constant_values=NG)
    batch3d = batchp.reshape(NBLK, 1, BN)

    xs = []
    for ln in ("l1", "l2", "l3"):
        p = params[ln]
        fin = p["Wq"].shape[0]
        kpad = 64 if fin == 55 else fin
        w4 = jnp.concatenate([p["Wq"], p["Wk"], p["Wv"], p["Wskip"]], axis=1)
        w4 = jnp.pad(w4, ((0, kpad - fin), (0, 0)))
        b4 = jnp.concatenate([p["bq"], p["bk"], p["bv"], p["bskip"]])
        b4 = b4.reshape(1, 4 * HC)
        q, kv, skip = _proj(x, w4, b4)
        e = _eproj(attr, p["We"])
        msg = mp(q, kv, e, src, dst)
        li = len(xs) + 1
        xcomb, st = _combine(msg, skip)
        x = _apply(xcomb, st,
                   params[f"bn{li}_g"].reshape(1, HC),
                   params[f"bn{li}_b"].reshape(1, HC))
        xs.append(x)

    sums = _pool(xs[0], xs[1], xs[2], batch3d)
    out = _final(sums, params["fc_W"],
                 params["fc_b"].reshape(1, 64),
                 params["bn_out_g"].reshape(1, 3 * HC),
                 params["bn_out_b"].reshape(1, 3 * HC))
    return out


# speedup vs baseline: 37.1602x; 1.7447x over previous
import jax
import jax.numpy as jnp
from jax import lax
from jax.experimental import pallas as pl
from jax.experimental.pallas import tpu as pltpu
from jax.experimental.pallas import tpu_sc as plsc

H = 5
C = 32
HC = H * C
NG = 16
N = 50000
E = 800000
NPAD = 50176
R = 16
RS = NPAD // R
ACC_ROWS = RS + 64
NSUB = 16
EPT = 51200
EPAD = NSUB * EPT
CH = 1600
NCH = EPT // CH
B = 32
PCAP = 6464
MSG_W = HC + 16
INV_SQRT_C = 1.0 / (C ** 0.5)


def _mp_body(qh, kvh, eh, srcv, dstv, out,
             pend_l, pend_s, pend_e,
             dbuf0, dbuf1, sbuf0, sbuf1,
             blk_g0, blk_g1, blk_l0, blk_l1, blk_s0, blk_s1,
             blk_e0, blk_e1, sblk0, sblk1,
             qbuf0, qbuf1, kvbuf0, kvbuf1, ebuf0, ebuf1,
             wvbuf0, wvbuf1, wbuf, acc,
             sem_g0, sem_g1, sem_s0, sem_s1, sem_c0, sem_c1):
    c = lax.axis_index("c")
    s = lax.axis_index("s")
    ebase = s * EPT
    iota = jnp.arange(16, dtype=jnp.int32)
    zeros16 = jnp.zeros((16,), jnp.float32)
    ones16i = jnp.ones((16,), jnp.int32)
    dbuf = (dbuf0, dbuf1)
    sbuf = (sbuf0, sbuf1)
    blk_g = (blk_g0, blk_g1)
    blk_l = (blk_l0, blk_l1)
    blk_s = (blk_s0, blk_s1)
    blk_e = (blk_e0, blk_e1)
    sblk = (sblk0, sblk1)
    qbuf = (qbuf0, qbuf1)
    kvbuf = (kvbuf0, kvbuf1)
    ebuf = (ebuf0, ebuf1)
    wvbuf = (wvbuf0, wvbuf1)
    sem_g = (sem_g0, sem_g1)
    sem_s = (sem_s0, sem_s1)
    sem_c = (sem_c0, sem_c1)

    def fire_chunk(ci, p):
        off = ebase + ci * CH
        pltpu.async_copy(dstv.at[pl.ds(off, CH)], dbuf[p], sem_c[p])
        pltpu.async_copy(srcv.at[pl.ds(off, CH)], sbuf[p], sem_c[p])

    def wait_chunk(ci, p):
        off = ebase + ci * CH
        pltpu.make_async_copy(dstv.at[pl.ds(off, CH)], dbuf[p], sem_c[p]).wait()
        pltpu.make_async_copy(srcv.at[pl.ds(off, CH)], sbuf[p], sem_c[p]).wait()

    def sweep_body(sweep, _):
        nb = (sweep * 2 + c) * RS

        def zw(i, _):
            wvbuf0[i // 11, pl.ds((i % 11) * 16, 16)] = zeros16
            return 0
        lax.fori_loop(0, B * 11, zw, 0)
        zbase = s * (ACC_ROWS // 16)
        for j in range(6):
            pltpu.sync_copy(wvbuf0.at[pl.ds(0, B)],
                            acc.at[pl.ds(zbase + j * B, B)])
        pltpu.sync_copy(wvbuf0.at[pl.ds(0, 8)],
                        acc.at[pl.ds(zbase + 192, 8)])
        plsc.subcore_barrier()

        def g_factory(p, off):
            def g_body(g, cntv):
                dvec = dbuf[p][pl.ds(g * 16, 16)]
                svec = sbuf[p][pl.ds(g * 16, 16)]
                evec = off + g * 16 + iota
                m = (dvec >= nb) & (dvec < nb + RS)
                pref = plsc.cumsum(ones16i, mask=m)
                pos = cntv + pref - 1
                plsc.store_scatter(pend_l, [pos], dvec - nb, mask=m)
                plsc.store_scatter(pend_s, [pos], svec, mask=m)
                plsc.store_scatter(pend_e, [pos], evec, mask=m)
                return cntv + plsc.all_reduce_population_count(m)
            return g_body

        fire_chunk(0, 0)

        def chunk_pair(it, cntv):
            for p in range(2):
                ci = 2 * it + p
                wait_chunk(ci, p)

                @pl.when(ci + 1 < NCH)
                def _():
                    fire_chunk(ci + 1, 1 - p)
                cntv = lax.fori_loop(0, CH // 16,
                                     g_factory(p, ebase + ci * CH), cntv)
            return cntv

        cntv = lax.fori_loop(0, NCH // 2, chunk_pair,
                             jnp.zeros((16,), jnp.int32))
        cnt = cntv[0]

        padl = jnp.full((16,), RS, jnp.int32)
        padz = jnp.zeros((16,), jnp.int32)
        for t in range(4):
            pend_l[pl.ds(cnt + t * 16, 16)] = padl
            pend_s[pl.ds(cnt + t * 16, 16)] = padz
            pend_e[pl.ds(cnt + t * 16, 16)] = padz
        nblocks = jnp.maximum(2, ((cnt + 2 * B - 1) // (2 * B)) * 2)

        def prep_fire(b, p):
            off = b * B
            for j in range(B // 16):
                plv = pend_l[pl.ds(off + j * 16, 16)]
                psv = pend_s[pl.ds(off + j * 16, 16)]
                pev = pend_e[pl.ds(off + j * 16, 16)]
                plc = jnp.clip(plv, 0, RS)
                blk_g[p][pl.ds(j * 16, 16)] = jnp.where(plc >= RS, 0, plc) + nb
                blk_l[p][pl.ds(j * 16, 16)] = plc
                blk_s[p][pl.ds(j * 16, 16)] = jnp.clip(psv, 0, NPAD - 1)
                blk_e[p][pl.ds(j * 16, 16)] = jnp.clip(pev, 0, E - 1)
            pltpu.async_copy(qh.at[blk_g[p]], qbuf[p], sem_g[p])
            pltpu.async_copy(kvh.at[blk_s[p]], kvbuf[p], sem_g[p])
            pltpu.async_copy(eh.at[blk_e[p]], ebuf[p], sem_g[p])

        m15 = iota == 15

        def compute(p):
            def arow2(b2):
                for h in range(H):
                    co = h * C
                    u = (qbuf[p][b2, pl.ds(co, 16)]
                         * (kvbuf[p][b2, pl.ds(co, 16)]
                            + ebuf[p][b2, pl.ds(co, 16)])
                         + qbuf[p][b2, pl.ds(co + 16, 16)]
                         * (kvbuf[p][b2, pl.ds(co + 16, 16)]
                            + ebuf[p][b2, pl.ds(co + 16, 16)]))
                    cs = plsc.cumsum(u)
                    plsc.store_scatter(
                        wbuf, [jnp.full((16,), h * B, jnp.int32) + b2], cs,
                        mask=m15)
            plsc.parallel_loop(0, B, unroll=2)(arow2)
            for i in range(H * B // 16):
                av = wbuf[pl.ds(i * 16, 16)]
                wbuf[pl.ds(i * 16, 16)] = jnp.exp(av * INV_SQRT_C)

            def wrow2(b2):
                den = zeros16
                for h in range(H):
                    wb = plsc.load_gather(
                        wbuf, [jnp.full((16,), h * B, jnp.int32) + b2])
                    den = den + jnp.where(iota == h, wb, 0.0)
                    for cc in range(2):
                        co = h * C + cc * 16
                        vv = (kvbuf[p][b2, pl.ds(HC + co, 16)]
                              + ebuf[p][b2, pl.ds(co, 16)])
                        wvbuf[p][b2, pl.ds(co, 16)] = wb * vv
                wvbuf[p][b2, pl.ds(HC, 16)] = den
            plsc.parallel_loop(0, B, unroll=2)(wrow2)

        prep_fire(0, 0)
        prep_fire(1, 1)

        def flush_pair(it, _):
            for p in range(2):
                b = 2 * it + p
                pltpu.make_async_copy(qh.at[blk_g[p]], qbuf[p],
                                      sem_g[p]).wait()
                pltpu.make_async_copy(kvh.at[blk_s[p]], kvbuf[p],
                                      sem_g[p]).wait()
                pltpu.make_async_copy(eh.at[blk_e[p]], ebuf[p],
                                      sem_g[p]).wait()

                @pl.when(it >= 1)
                def _():
                    pltpu.make_async_copy(wvbuf[p], acc.at[sblk[p]],
                                          sem_s[p]).wait()
                compute(p)
                for j in range(B // 16):
                    sblk[p][pl.ds(j * 16, 16)] = blk_l[p][pl.ds(j * 16, 16)]
                pltpu.async_copy(wvbuf[p], acc.at[sblk[p]], sem_s[p],
                                 add=True)

                @pl.when(b + 2 < nblocks)
                def _():
                    prep_fire(b + 2, p)
            return 0

        lax.fori_loop(0, nblocks // 2, flush_pair, 0)
        for p in range(2):
            pltpu.make_async_copy(wvbuf[p], acc.at[sblk[p]], sem_s[p]).wait()
        plsc.subcore_barrier()

        @pl.when(s < 8)
        def _():
            pltpu.sync_copy(acc.at[pl.ds(s * 392, 392)],
                            out.at[pl.ds(nb + s * 392, 392)])
        plsc.subcore_barrier()
        return 0

    lax.fori_loop(0, R // 2, sweep_body, 0)


def _make_mp():
    mesh = plsc.VectorSubcoreMesh(core_axis_name="c", subcore_axis_name="s")
    i32 = jnp.int32
    f32 = jnp.float32
    return pl.kernel(
        _mp_body,
        out_type=jax.ShapeDtypeStruct((NPAD, MSG_W), f32),
        mesh=mesh,
        compiler_params=pltpu.CompilerParams(needs_layout_passes=False,
                                             use_tc_tiling_on_sc=False),
        scratch_types=(
            [pltpu.VMEM((PCAP,), i32)] * 3
            + [pltpu.VMEM((CH,), i32)] * 4
            + [pltpu.VMEM((B,), i32)] * 10
            + [pltpu.VMEM((B, HC), f32)] * 2
            + [pltpu.VMEM((B, 2 * HC), f32)] * 2
            + [pltpu.VMEM((B, HC), f32)] * 2
            + [pltpu.VMEM((B, MSG_W), f32)] * 2
            + [pltpu.VMEM((H * B,), f32)]
            + [pltpu.VMEM_SHARED((ACC_ROWS, MSG_W), f32)]
            + [pltpu.SemaphoreType.DMA] * 6
        ),
    )


BN = 1792
NBLK = NPAD // BN
BE = 4000


def _proj_fn(x_ref, w_ref, b_ref, q_ref, kv_ref, s_ref):
    y = jnp.dot(x_ref[...], w_ref[...],
                preferred_element_type=jnp.float32) + b_ref[...]
    q_ref[...] = y[:, 0 * HC:1 * HC]
    kv_ref[...] = y[:, 1 * HC:3 * HC]
    s_ref[...] = y[:, 3 * HC:4 * HC]


def _proj(x, w4, b4):
    k = x.shape[1]
    outs = [jax.ShapeDtypeStruct((NPAD, HC), jnp.float32),
            jax.ShapeDtypeStruct((NPAD, 2 * HC), jnp.float32),
            jax.ShapeDtypeStruct((NPAD, HC), jnp.float32)]
    return pl.pallas_call(
        _proj_fn,
        grid=(NBLK,),
        in_specs=[
            pl.BlockSpec((BN, k), lambda i: (i, 0)),
            pl.BlockSpec((k, 4 * HC), lambda i: (0, 0)),
            pl.BlockSpec((1, 4 * HC), lambda i: (0, 0)),
        ],
        out_specs=[pl.BlockSpec((BN, HC), lambda i: (i, 0)),
                   pl.BlockSpec((BN, 2 * HC), lambda i: (i, 0)),
                   pl.BlockSpec((BN, HC), lambda i: (i, 0))],
        out_shape=outs,
    )(x, w4, b4)


def _eproj_fn(a_ref, w_ref, o_ref):
    o_ref[...] = jnp.dot(a_ref[...], w_ref[...],
                         preferred_element_type=jnp.float32)


def _eproj(attr, we):
    return pl.pallas_call(
        _eproj_fn,
        grid=(E // BE,),
        in_specs=[
            pl.BlockSpec((BE, 16), lambda i: (i, 0)),
            pl.BlockSpec((16, HC), lambda i: (0, 0)),
        ],
        out_specs=pl.BlockSpec((BE, HC), lambda i: (i, 0)),
        out_shape=jax.ShapeDtypeStruct((E, HC), jnp.float32),
    )(attr, we)


def _combine_fn(msg_ref, skip_ref, x_ref, st_ref):
    pid = pl.program_id(0)
    m = msg_ref[...]
    den = m[:, HC:HC + H]
    den = jnp.repeat(den, C, axis=1)
    x = m[:, :HC] / (den + 1e-16) + skip_ref[...]
    row = pid * BN + lax.broadcasted_iota(jnp.int32, (BN, 1), 0)
    x = jnp.where(row < N, x, 0.0)
    x_ref[...] = x

    @pl.when(pid == 0)
    def _():
        st_ref[...] = jnp.zeros_like(st_ref)
    upd = jnp.concatenate(
        [jnp.sum(x, axis=0, keepdims=True),
         jnp.sum(x * x, axis=0, keepdims=True),
         jnp.zeros((6, HC), jnp.float32)], axis=0)
    st_ref[...] += upd


def _combine(msg, skip):
    return pl.pallas_call(
        _combine_fn,
        grid=(NBLK,),
        in_specs=[
            pl.BlockSpec((BN, MSG_W), lambda i: (i, 0)),
            pl.BlockSpec((BN, HC), lambda i: (i, 0)),
        ],
        out_specs=[
            pl.BlockSpec((BN, HC), lambda i: (i, 0)),
            pl.BlockSpec((8, HC), lambda i: (0, 0)),
        ],
        out_shape=[
            jax.ShapeDtypeStruct((NPAD, HC), jnp.float32),
            jax.ShapeDtypeStruct((8, HC), jnp.float32),
        ],
    )(msg, skip)


def _apply_fn(x_ref, st_ref, g_ref, b_ref, o_ref):
    pid = pl.program_id(0)
    st = st_ref[...]
    mean = st[0:1, :] / float(N)
    var = st[1:2, :] / float(N) - mean * mean
    rstd = lax.rsqrt(var + 1e-5)
    y = g_ref[...] * (x_ref[...] - mean) * rstd + b_ref[...]
    y = jnp.maximum(y, 0.0)
    row = pid * BN + lax.broadcasted_iota(jnp.int32, (BN, 1), 0)
    o_ref[...] = jnp.where(row < N, y, 0.0)


def _apply(x, st, g, b):
    return pl.pallas_call(
        _apply_fn,
        grid=(NBLK,),
        in_specs=[
            pl.BlockSpec((BN, HC), lambda i: (i, 0)),
            pl.BlockSpec((8, HC), lambda i: (0, 0)),
            pl.BlockSpec((1, HC), lambda i: (0, 0)),
            pl.BlockSpec((1, HC), lambda i: (0, 0)),
        ],
        out_specs=pl.BlockSpec((BN, HC), lambda i: (i, 0)),
        out_shape=jax.ShapeDtypeStruct((NPAD, HC), jnp.float32),
    )(x, st, g, b)


def _pool_fn(x1_ref, x2_ref, x3_ref, b_ref, s_ref):
    pid = pl.program_id(0)
    bv = b_ref[0, 0, :]
    oh = (lax.broadcasted_iota(jnp.int32, (NG, BN), 0)
          == bv[None, :]).astype(jnp.float32)
    xc = jnp.concatenate(
        [x1_ref[...], x2_ref[...], x3_ref[...],
         jnp.ones((BN, 128), jnp.float32)], axis=1)
    acc = jnp.dot(oh, xc, preferred_element_type=jnp.float32)

    @pl.when(pid == 0)
    def _():
        s_ref[...] = jnp.zeros_like(s_ref)
    s_ref[...] += acc


def _pool(x1, x2, x3, batch3d):
    return pl.pallas_call(
        _pool_fn,
        grid=(NBLK,),
        in_specs=[
            pl.BlockSpec((BN, HC), lambda i: (i, 0)),
            pl.BlockSpec((BN, HC), lambda i: (i, 0)),
            pl.BlockSpec((BN, HC), lambda i: (i, 0)),
            pl.BlockSpec((1, 1, BN), lambda i: (i, 0, 0)),
        ],
        out_specs=pl.BlockSpec((NG, 3 * HC + 128), lambda i: (0, 0)),
        out_shape=jax.ShapeDtypeStruct((NG, 3 * HC + 128), jnp.float32),
    )(x1, x2, x3, batch3d)


def _final_fn(s_ref, w_ref, b_ref, g_ref, bb_ref, o_ref):
    srow = s_ref[...]
    cnt = jnp.maximum(srow[:, 3 * HC:3 * HC + 1], 1.0)
    pooled = srow[:, :3 * HC] / cnt
    m = jnp.mean(pooled, axis=0, keepdims=True)
    v = jnp.mean((pooled - m) ** 2, axis=0, keepdims=True)
    pn = g_ref[...] * (pooled - m) * lax.rsqrt(v + 1e-5) + bb_ref[...]
    o_ref[...] = jnp.dot(pn, w_ref[...],
                         preferred_element_type=jnp.float32) + b_ref[...]


def _final(sums, fcw, fcb, g, b):
    return pl.pallas_call(
        _final_fn,
        in_specs=[pl.BlockSpec(sums.shape, lambda: (0, 0)),
                  pl.BlockSpec(fcw.shape, lambda: (0, 0)),
                  pl.BlockSpec(fcb.shape, lambda: (0, 0)),
                  pl.BlockSpec(g.shape, lambda: (0, 0)),
                  pl.BlockSpec(b.shape, lambda: (0, 0))],
        out_specs=pl.BlockSpec((NG, 64), lambda: (0, 0)),
        out_shape=jax.ShapeDtypeStruct((NG, 64), jnp.float32),
    )(sums, fcw, fcb, g, b)


def kernel(X, idx, attr, batch, params):
    mp = _make_mp()

    src = jnp.pad(idx[0], (0, EPAD - E))
    dst = jnp.pad(idx[1], (0, EPAD - E), constant_values=-1)
    x = jnp.pad(X, ((0, NPAD - N), (0, 64 - X.shape[1])))
    batchp = jnp.pad(batch, (0, NPAD - N), constant_values=NG)
    batch3d = batchp.reshape(NBLK, 1, BN)

    xs = []
    for ln in ("l1", "l2", "l3"):
        p = params[ln]
        fin = p["Wq"].shape[0]
        kpad = 64 if fin == 55 else fin
        w4 = jnp.concatenate([p["Wq"], p["Wk"], p["Wv"], p["Wskip"]], axis=1)
        w4 = jnp.pad(w4, ((0, kpad - fin), (0, 0)))
        b4 = jnp.concatenate([p["bq"], p["bk"], p["bv"], p["bskip"]])
        b4 = b4.reshape(1, 4 * HC)
        q, kv, skip = _proj(x, w4, b4)
        e = _eproj(attr, p["We"])
        msg = mp(q, kv, e, src, dst)
        li = len(xs) + 1
        xcomb, st = _combine(msg, skip)
        x = _apply(xcomb, st,
                   params[f"bn{li}_g"].reshape(1, HC),
                   params[f"bn{li}_b"].reshape(1, HC))
        xs.append(x)

    sums = _pool(xs[0], xs[1], xs[2], batch3d)
    out = _final(sums, params["fc_W"],
                 params["fc_b"].reshape(1, 64),
                 params["bn_out_g"].reshape(1, 3 * HC),
                 params["bn_out_b"].reshape(1, 3 * HC))
    return out
